# baseline (device time: 97124 ns/iter reference)
import jax
import jax.numpy as jnp
from jax import lax
from jax.experimental import pallas as pl
from jax.experimental.pallas import tpu as pltpu

N_DEV = 16
M_PER = 256
N_COLS = 2048

GROUPS_PER_DIR = 4
RING_CFG = []
for _g in range(GROUPS_PER_DIR):
    RING_CFG.append(+1)
    RING_CFG.append(-1)
RING_W = N_COLS // len(RING_CFG)

COMM_DTYPE = jnp.bfloat16


def kernel(x, w_mat, scale_x, scale_w):
    m_global, k_per = x.shape
    _, n = w_mat.shape

    def body(x_ref, w_ref, sx_ref, sw_ref, out_ref, *scr):
        rings = []
        for j, dirn in enumerate(RING_CFG):
            sb, rb, ss, rs, cr = scr[5 * j:5 * j + 5]
            rings.append((dirn, j * RING_W, sb, rb, ss, rs, cr))

        my = lax.axis_index("i")
        left = lax.rem(my + N_DEV - 1, N_DEV)
        right = lax.rem(my + 1, N_DEV)

        barrier_sem = pltpu.get_barrier_semaphore()
        for nbr in (left, right):
            pl.semaphore_signal(
                barrier_sem, inc=1,
                device_id=(nbr,), device_id_type=pl.DeviceIdType.MESH,
            )
        pl.semaphore_wait(barrier_sem, 2)

        def partial_chunk(c, off):
            xc = x_ref[pl.ds(c * M_PER, M_PER), :]
            return lax.dot_general(
                xc, w_ref[:, off:off + RING_W],
                dimension_numbers=(((1,), (0,)), ((), ())),
                preferred_element_type=jnp.int32,
            )

        descs = {}
        for h in range(N_DEV - 1):
            for j, (dirn, off, sb, rb, ss, rs, cr) in enumerate(rings):
                dst = lax.rem(my + dirn + N_DEV, N_DEV)
                src = lax.rem(my - dirn + N_DEV, N_DEV)
                c = lax.rem(my - dirn * (h + 1) + 2 * N_DEV, N_DEV)
                part = partial_chunk(c, off)
                if h == 0:
                    val = part.astype(COMM_DTYPE)
                else:
                    descs[(j, h - 1)].wait_recv()
                    if COMM_DTYPE == jnp.int32:
                        val = part + rb[(h - 1) % 2, :, :]
                    else:
                        val = (
                            part.astype(jnp.float32)
                            + rb[(h - 1) % 2, :, :].astype(jnp.float32)
                        ).astype(COMM_DTYPE)
                    if h + 1 <= N_DEV - 2:
                        pl.semaphore_signal(
                            cr, inc=1,
                            device_id=(src,),
                            device_id_type=pl.DeviceIdType.MESH,
                        )
                if h >= 2:
                    descs[(j, h - 2)].wait_send()
                    pl.semaphore_wait(cr, 1)
                sb[h % 2, :, :] = val
                rdma = pltpu.make_async_remote_copy(
                    src_ref=sb.at[h % 2],
                    dst_ref=rb.at[h % 2],
                    send_sem=ss.at[h % 2],
                    recv_sem=rs.at[h % 2],
                    device_id=(dst,),
                    device_id_type=pl.DeviceIdType.MESH,
                )
                rdma.start()
                descs[(j, h)] = rdma

        s = sx_ref[0] * sw_ref[0]
        for j, (dirn, off, sb, rb, ss, rs, cr) in enumerate(rings):
            descs[(j, N_DEV - 2)].wait_recv()
            acc = (
                partial_chunk(my, off).astype(jnp.float32)
                + rb[(N_DEV - 2) % 2, :, :].astype(jnp.float32)
            )
            out_ref[:, off:off + RING_W] = acc * s
            descs[(j, N_DEV - 3)].wait_send()
            descs[(j, N_DEV - 2)].wait_send()

    scratch = []
    for _ in RING_CFG:
        scratch += [
            pltpu.VMEM((2, M_PER, RING_W), COMM_DTYPE),
            pltpu.VMEM((2, M_PER, RING_W), COMM_DTYPE),
            pltpu.SemaphoreType.DMA((2,)),
            pltpu.SemaphoreType.DMA((2,)),
            pltpu.SemaphoreType.REGULAR,
        ]

    return pl.pallas_call(
        body,
        out_shape=jax.ShapeDtypeStruct((M_PER, n), jnp.float32),
        in_specs=[
            pl.BlockSpec(memory_space=pltpu.VMEM),
            pl.BlockSpec(memory_space=pltpu.VMEM),
            pl.BlockSpec(memory_space=pltpu.SMEM),
            pl.BlockSpec(memory_space=pltpu.SMEM),
        ],
        out_specs=pl.BlockSpec(memory_space=pltpu.VMEM),
        scratch_shapes=scratch,
        compiler_params=pltpu.CompilerParams(collective_id=0),
    )(x, w_mat, scale_x, scale_w)


# device time: 96980 ns/iter; 1.0015x vs baseline; 1.0015x over previous
import jax
import jax.numpy as jnp
from jax import lax
from jax.experimental import pallas as pl
from jax.experimental.pallas import tpu as pltpu

N_DEV = 16
M_PER = 256
N_COLS = 2048
N_HOPS = N_DEV - 1

GROUPS_PER_DIR = 2
RING_CFG = []
for _g in range(GROUPS_PER_DIR):
    RING_CFG.append(+1)
    RING_CFG.append(-1)
RING_W = N_COLS // len(RING_CFG)

NSLOTS = 2
COMM_DTYPE = jnp.bfloat16


def kernel(x, w_mat, scale_x, scale_w):
    m_global, k_per = x.shape
    _, n = w_mat.shape

    def body(x_ref, w_ref, sx_ref, sw_ref, out_ref, xb_ref, wb_ref, *scr):
        rings = []
        for j, dirn in enumerate(RING_CFG):
            sb, rb, ss, rs, cr = scr[5 * j:5 * j + 5]
            rings.append((dirn, j * RING_W, sb, rb, ss, rs, cr))

        my = lax.axis_index("i")
        left = lax.rem(my + N_DEV - 1, N_DEV)
        right = lax.rem(my + 1, N_DEV)

        barrier_sem = pltpu.get_barrier_semaphore()
        for nbr in (left, right):
            pl.semaphore_signal(
                barrier_sem, inc=1,
                device_id=(nbr,), device_id_type=pl.DeviceIdType.MESH,
            )
        pl.semaphore_wait(barrier_sem, 2)

        xb_ref[:, :] = x_ref[:, :].astype(COMM_DTYPE)
        wb_ref[:, :] = w_ref[:, :].astype(COMM_DTYPE)

        def partial_chunk(c, off):
            xc = xb_ref[pl.ds(c * M_PER, M_PER), :]
            return lax.dot_general(
                xc, wb_ref[:, off:off + RING_W],
                dimension_numbers=(((1,), (0,)), ((), ())),
                preferred_element_type=jnp.float32,
            )

        descs = {}
        for h in range(N_HOPS):
            for j, (dirn, off, sb, rb, ss, rs, cr) in enumerate(rings):
                dst = lax.rem(my + dirn + N_DEV, N_DEV)
                src = lax.rem(my - dirn + N_DEV, N_DEV)
                c = lax.rem(my - dirn * (h + 1) + 2 * N_DEV, N_DEV)
                part = partial_chunk(c, off).astype(COMM_DTYPE)
                if h == 0:
                    val = part
                else:
                    descs[(j, h - 1)].wait_recv()
                    val = part + rb[(h - 1) % NSLOTS, :, :]
                    if h - 1 + NSLOTS <= N_HOPS - 1:
                        pl.semaphore_signal(
                            cr, inc=1,
                            device_id=(src,),
                            device_id_type=pl.DeviceIdType.MESH,
                        )
                if h >= NSLOTS:
                    descs[(j, h - NSLOTS)].wait_send()
                    pl.semaphore_wait(cr, 1)
                sb[h % NSLOTS, :, :] = val
                rdma = pltpu.make_async_remote_copy(
                    src_ref=sb.at[h % NSLOTS],
                    dst_ref=rb.at[h % NSLOTS],
                    send_sem=ss.at[h % NSLOTS],
                    recv_sem=rs.at[h % NSLOTS],
                    device_id=(dst,),
                    device_id_type=pl.DeviceIdType.MESH,
                )
                rdma.start()
                descs[(j, h)] = rdma

        s = sx_ref[0] * sw_ref[0]
        for j, (dirn, off, sb, rb, ss, rs, cr) in enumerate(rings):
            descs[(j, N_HOPS - 1)].wait_recv()
            acc = (
                partial_chunk(my, off)
                + rb[(N_HOPS - 1) % NSLOTS, :, :].astype(jnp.float32)
            )
            out_ref[:, off:off + RING_W] = acc * s
            for hh in range(max(0, N_HOPS - NSLOTS), N_HOPS):
                descs[(j, hh)].wait_send()

    scratch = []
    for _ in RING_CFG:
        scratch += [
            pltpu.VMEM((NSLOTS, M_PER, RING_W), COMM_DTYPE),
            pltpu.VMEM((NSLOTS, M_PER, RING_W), COMM_DTYPE),
            pltpu.SemaphoreType.DMA((NSLOTS,)),
            pltpu.SemaphoreType.DMA((NSLOTS,)),
            pltpu.SemaphoreType.REGULAR,
        ]

    return pl.pallas_call(
        body,
        out_shape=jax.ShapeDtypeStruct((M_PER, n), jnp.float32),
        in_specs=[
            pl.BlockSpec(memory_space=pltpu.VMEM),
            pl.BlockSpec(memory_space=pltpu.VMEM),
            pl.BlockSpec(memory_space=pltpu.SMEM),
            pl.BlockSpec(memory_space=pltpu.SMEM),
        ],
        out_specs=pl.BlockSpec(memory_space=pltpu.VMEM),
        scratch_shapes=[
            pltpu.VMEM((m_global, k_per), COMM_DTYPE),
            pltpu.VMEM((k_per, n), COMM_DTYPE),
        ] + scratch,
        compiler_params=pltpu.CompilerParams(collective_id=0),
    )(x, w_mat, scale_x, scale_w)
